# baseline (speedup 1.0000x reference)
import jax
import jax.numpy as jnp
from jax import lax
from jax.experimental import pallas as pl
from jax.experimental.pallas import tpu as pltpu
from jax.experimental.pallas import tpu_sc as plsc

MAXLEN = 200
EMBED = 64
ROWW = 2 * EMBED

_info = plsc.get_sparse_core_info()
NC, NS = _info.num_cores, _info.num_subcores
NW = NC * NS

SEQ_PER_CHUNK = 2
CR = SEQ_PER_CHUNK * MAXLEN
NBUF = 4


def _body(x_hbm, tbl_hbm, pos_hbm, out_hbm, idx_v, rows_v, gsem, osem, psem):
    wid = lax.axis_index("s") * NC + lax.axis_index("c")
    n_flat = x_hbm.shape[0]
    per_w = n_flat // NW
    n_chunks = per_w // CR
    base = wid * per_w

    def prep(g, s):
        row0 = base + g * CR
        pltpu.async_copy(x_hbm.at[pl.ds(row0, CR)], idx_v[s], psem[s])
        for q in range(SEQ_PER_CHUNK):
            pltpu.async_copy(
                pos_hbm, rows_v[s].at[pl.ds(q * MAXLEN, MAXLEN)], psem[s]
            )

    def wait_prep(s):
        pltpu.make_async_copy(x_hbm.at[pl.ds(base, CR)], idx_v[s], psem[s]).wait()
        for q in range(SEQ_PER_CHUNK):
            pltpu.make_async_copy(
                pos_hbm, rows_v[s].at[pl.ds(q * MAXLEN, MAXLEN)], psem[s]
            ).wait()

    def fire_gather(s):
        pltpu.async_copy(tbl_hbm.at[idx_v[s]], rows_v[s], gsem[s], add=True)

    def out_slice(g):
        return out_hbm.at[pl.ds(base + g * CR, CR), pl.ds(0, EMBED)]

    def wait_out(s):
        pltpu.make_async_copy(rows_v[s], out_slice(0), osem[s]).wait()

    prep(0, 0)
    prep(1, 1)
    wait_prep(0)
    fire_gather(0)

    def quad_body(t, carry):
        for b in range(NBUF):
            g = NBUF * t + b
            s1, s2 = (b + 1) % NBUF, (b + 2) % NBUF
            pltpu.make_async_copy(tbl_hbm.at[idx_v[b]], rows_v[b], gsem[b]).wait()
            pltpu.async_copy(rows_v[b], out_slice(g), osem[b])

            @pl.when(g + 2 < n_chunks)
            def _():
                @pl.when(g >= 2)
                def _():
                    wait_out(s2)
                prep(g + 2, s2)

            @pl.when(g + 1 < n_chunks)
            def _():
                wait_prep(s1)
                fire_gather(s1)
        return carry

    lax.fori_loop(0, n_chunks // NBUF, quad_body, 0, unroll=False)
    for b in range(NBUF):
        wait_out(b)


@jax.jit
def kernel(x, token_table, pos_table):
    batch, seq_len = x.shape
    n_flat = batch * seq_len
    vocab = token_table.shape[0]
    x_flat = x.reshape(n_flat).astype(jnp.int32)
    tbl = lax.optimization_barrier(token_table.reshape(vocab // 2, ROWW))
    tbl = tbl.reshape(vocab, EMBED)

    mesh = plsc.VectorSubcoreMesh(core_axis_name="c", subcore_axis_name="s")
    run = pl.kernel(
        _body,
        out_type=jax.ShapeDtypeStruct((n_flat, ROWW), jnp.float32),
        mesh=mesh,
        scratch_types=[
            [pltpu.VMEM((CR,), jnp.int32) for _ in range(NBUF)],
            [pltpu.VMEM((CR, EMBED), jnp.float32) for _ in range(NBUF)],
            [pltpu.SemaphoreType.DMA for _ in range(NBUF)],
            [pltpu.SemaphoreType.DMA for _ in range(NBUF)],
            [pltpu.SemaphoreType.DMA for _ in range(NBUF)],
        ],
        compiler_params=pltpu.CompilerParams(use_tc_tiling_on_sc=False),
    )
    out = run(x_flat, tbl, pos_table)
    return out[:, :EMBED].reshape(batch, seq_len, EMBED)


# speedup vs baseline: 1.0005x; 1.0005x over previous
import jax
import jax.numpy as jnp
from jax import lax
from jax.experimental import pallas as pl
from jax.experimental.pallas import tpu as pltpu
from jax.experimental.pallas import tpu_sc as plsc

MAXLEN = 200
EMBED = 64
ROWW = 2 * EMBED

_info = plsc.get_sparse_core_info()
NC, NS = _info.num_cores, _info.num_subcores
NW = NC * NS

SEQ_PER_CHUNK = 2
CR = SEQ_PER_CHUNK * MAXLEN
NBUF = 4


def _body(x_hbm, tbl_hbm, pos_hbm, out_hbm, idx_v, rows_v, gsem, osem, psem):
    wid = lax.axis_index("s") * NC + lax.axis_index("c")
    n_flat = x_hbm.shape[0]
    per_w = n_flat // NW
    n_chunks = per_w // CR
    base = wid * per_w

    def prep(g, s):
        row0 = base + g * CR
        pltpu.async_copy(x_hbm.at[pl.ds(row0, CR)], idx_v[s], psem[s])
        for q in range(SEQ_PER_CHUNK):
            pltpu.async_copy(
                pos_hbm, rows_v[s].at[pl.ds(q * MAXLEN, MAXLEN)], psem[s]
            )

    def wait_prep(s):
        pltpu.make_async_copy(x_hbm.at[pl.ds(base, CR)], idx_v[s], psem[s]).wait()
        for q in range(SEQ_PER_CHUNK):
            pltpu.make_async_copy(
                pos_hbm, rows_v[s].at[pl.ds(q * MAXLEN, MAXLEN)], psem[s]
            ).wait()

    def fire_gather(s):
        pltpu.async_copy(tbl_hbm.at[idx_v[s]], rows_v[s], gsem[s], add=True)

    def out_slice(g):
        return out_hbm.at[pl.ds(base + g * CR, CR), pl.ds(0, EMBED)]

    def wait_out(s):
        pltpu.make_async_copy(rows_v[s], out_slice(0), osem[s]).wait()

    for g0 in range(3):
        prep(g0, g0)
    wait_prep(0)
    fire_gather(0)
    wait_prep(1)
    fire_gather(1)

    def quad_body(t, carry):
        for b in range(NBUF):
            g = NBUF * t + b
            s2, s3 = (b + 2) % NBUF, (b + 3) % NBUF
            pltpu.make_async_copy(tbl_hbm.at[idx_v[b]], rows_v[b], gsem[b]).wait()
            pltpu.async_copy(rows_v[b], out_slice(g), osem[b])

            @pl.when(g + 3 < n_chunks)
            def _():
                @pl.when(g >= 1)
                def _():
                    wait_out(s3)
                prep(g + 3, s3)

            @pl.when(g + 2 < n_chunks)
            def _():
                wait_prep(s2)
                fire_gather(s2)
        return carry

    lax.fori_loop(0, n_chunks // NBUF, quad_body, 0, unroll=False)
    for b in range(NBUF):
        wait_out(b)


@jax.jit
def kernel(x, token_table, pos_table):
    batch, seq_len = x.shape
    n_flat = batch * seq_len
    vocab = token_table.shape[0]
    x_flat = x.reshape(n_flat).astype(jnp.int32)
    tbl = lax.optimization_barrier(token_table.reshape(vocab // 2, ROWW))
    tbl = tbl.reshape(vocab, EMBED)

    mesh = plsc.VectorSubcoreMesh(core_axis_name="c", subcore_axis_name="s")
    run = pl.kernel(
        _body,
        out_type=jax.ShapeDtypeStruct((n_flat, ROWW), jnp.float32),
        mesh=mesh,
        scratch_types=[
            [pltpu.VMEM((CR,), jnp.int32) for _ in range(NBUF)],
            [pltpu.VMEM((CR, EMBED), jnp.float32) for _ in range(NBUF)],
            [pltpu.SemaphoreType.DMA for _ in range(NBUF)],
            [pltpu.SemaphoreType.DMA for _ in range(NBUF)],
            [pltpu.SemaphoreType.DMA for _ in range(NBUF)],
        ],
        compiler_params=pltpu.CompilerParams(use_tc_tiling_on_sc=False),
    )
    out = run(x_flat, tbl, pos_table)
    return out[:, :EMBED].reshape(batch, seq_len, EMBED)


# speedup vs baseline: 1.4421x; 1.4414x over previous
import jax
import jax.numpy as jnp
from jax import lax
from jax.experimental import pallas as pl
from jax.experimental.pallas import tpu as pltpu
from jax.experimental.pallas import tpu_sc as plsc

MAXLEN = 200
EMBED = 64
ROWW = 2 * EMBED

_info = plsc.get_sparse_core_info()
NC, NS = _info.num_cores, _info.num_subcores
NW = NC * NS

SEQ_PER_CHUNK = 2
CR = SEQ_PER_CHUNK * MAXLEN
NBUF = 4


def _body(x_hbm, tbl_hbm, posr_hbm, out_hbm, idx_v, rows_v, gsem, osem, psem):
    wid = lax.axis_index("s") * NC + lax.axis_index("c")
    n_flat = x_hbm.shape[0]
    per_w = n_flat // NW
    n_chunks = per_w // CR
    base = wid * per_w

    def prep(g, s):
        row0 = base + g * CR
        pltpu.async_copy(x_hbm.at[pl.ds(row0, CR)], idx_v[s], psem[s])
        pltpu.async_copy(posr_hbm.at[wid], rows_v[s], psem[s])

    def wait_prep(s):
        pltpu.make_async_copy(x_hbm.at[pl.ds(base, CR)], idx_v[s], psem[s]).wait()
        pltpu.make_async_copy(posr_hbm.at[wid], rows_v[s], psem[s]).wait()

    def fire_gather(s):
        pltpu.async_copy(tbl_hbm.at[idx_v[s]], rows_v[s], gsem[s], add=True)

    def out_slice(g):
        return out_hbm.at[pl.ds(base + g * CR, CR), pl.ds(0, EMBED)]

    def wait_out(s):
        pltpu.make_async_copy(rows_v[s], out_slice(0), osem[s]).wait()

    for g0 in range(3):
        prep(g0, g0)
    wait_prep(0)
    fire_gather(0)
    wait_prep(1)
    fire_gather(1)

    def quad_body(t, carry):
        for b in range(NBUF):
            g = NBUF * t + b
            s2, s3 = (b + 2) % NBUF, (b + 3) % NBUF
            pltpu.make_async_copy(tbl_hbm.at[idx_v[b]], rows_v[b], gsem[b]).wait()
            pltpu.async_copy(rows_v[b], out_slice(g), osem[b])

            @pl.when(g + 3 < n_chunks)
            def _():
                @pl.when(g >= 1)
                def _():
                    wait_out(s3)
                prep(g + 3, s3)

            @pl.when(g + 2 < n_chunks)
            def _():
                wait_prep(s2)
                fire_gather(s2)
        return carry

    lax.fori_loop(0, n_chunks // NBUF, quad_body, 0, unroll=False)
    for b in range(NBUF):
        wait_out(b)


@jax.jit
def kernel(x, token_table, pos_table):
    batch, seq_len = x.shape
    n_flat = batch * seq_len
    vocab = token_table.shape[0]
    x_flat = x.reshape(n_flat).astype(jnp.int32)
    tbl = lax.optimization_barrier(token_table.reshape(vocab // 2, ROWW))
    tbl = tbl.reshape(vocab, EMBED)

    mesh = plsc.VectorSubcoreMesh(core_axis_name="c", subcore_axis_name="s")
    run = pl.kernel(
        _body,
        out_type=jax.ShapeDtypeStruct((n_flat, ROWW), jnp.float32),
        mesh=mesh,
        scratch_types=[
            [pltpu.VMEM((CR,), jnp.int32) for _ in range(NBUF)],
            [pltpu.VMEM((CR, EMBED), jnp.float32) for _ in range(NBUF)],
            [pltpu.SemaphoreType.DMA for _ in range(NBUF)],
            [pltpu.SemaphoreType.DMA for _ in range(NBUF)],
            [pltpu.SemaphoreType.DMA for _ in range(NBUF)],
        ],
        compiler_params=pltpu.CompilerParams(use_tc_tiling_on_sc=False),
    )
    pos_rep = jnp.broadcast_to(
        jnp.tile(pos_table, (SEQ_PER_CHUNK, 1))[None], (NW, CR, EMBED)
    )
    out = run(x_flat, tbl, pos_rep)
    return out[:, :EMBED].reshape(batch, seq_len, EMBED)
